# baseline (speedup 1.0000x reference)
import functools

import jax
import jax.numpy as jnp
from jax import lax
from jax.experimental import pallas as pl
from jax.experimental.pallas import tpu as pltpu
from jax.experimental.pallas import tpu_sc as plsc

LN_EPS = 1e-12


_RBLK = 8192


_QBLK = _RBLK // 4
_HI_MASK = -65536


def _round_bf16_bits(t):
    bits = lax.bitcast_convert_type(t, jnp.int32)
    rnd = bits + 0x7FFF + ((bits >> 16) & 1)
    return rnd & _HI_MASK


def _relayout_body(x_ref, o_ref):
    e = x_ref.shape[0]
    for k in range(_QBLK // 128):
        sl = pl.ds(k * 128, 128)
        quads = [
            _round_bf16_bits(jnp.transpose(
                x_ref[:, q * _QBLK + k * 128:q * _QBLK + (k + 1) * 128]))
            for q in range(4)
        ]
        p01 = quads[0] | ((quads[1] >> 16) & 0xFFFF)
        p23 = quads[2] | ((quads[3] >> 16) & 0xFFFF)
        o_ref[sl, :e] = lax.bitcast_convert_type(p01, jnp.float32)
        o_ref[sl, e:] = lax.bitcast_convert_type(p23, jnp.float32)


def _tc_relayout(tableT):
    e, v = tableT.shape
    grid = (v + _RBLK - 1) // _RBLK
    return pl.pallas_call(
        _relayout_body,
        grid=(grid,),
        in_specs=[pl.BlockSpec((e, _RBLK), lambda j: (0, j))],
        out_specs=pl.BlockSpec((_QBLK, 2 * e), lambda j: (j, 0)),
        out_shape=jax.ShapeDtypeStruct((grid * _QBLK, 2 * e), jnp.float32),
        compiler_params=pltpu.CompilerParams(
            dimension_semantics=("parallel",)),
    )(tableT)


_CHUNK = 128


def _sc_gather(table, idx_flat):
    info = plsc.get_sparse_core_info()
    _NC, _NS = info.num_cores, info.num_subcores
    _NW = _NC * _NS
    n, e = idx_flat.shape[0], table.shape[1]
    per_w = n // _NW
    n_chunks = per_w // _CHUNK
    mesh = plsc.VectorSubcoreMesh(core_axis_name="c", subcore_axis_name="s")

    @functools.partial(
        pl.kernel,
        mesh=mesh,
        compiler_params=pltpu.CompilerParams(use_tc_tiling_on_sc=True),
        out_type=jax.ShapeDtypeStruct((n, e), jnp.float32),
        scratch_types=[
            pltpu.VMEM((n_chunks, _CHUNK), jnp.int32),
            pltpu.VMEM((n_chunks, _CHUNK, e), jnp.float32),
            pltpu.SemaphoreType.DMA,
        ],
    )
    def k(table_hbm, idx_hbm, out_hbm, idx_v, rows_v, sem):
        wid = lax.axis_index("s") * _NC + lax.axis_index("c")
        base = wid * per_w
        for j in range(n_chunks):
            pltpu.sync_copy(idx_hbm.at[pl.ds(base + j * _CHUNK, _CHUNK)],
                            idx_v.at[j])
        copies = [
            pltpu.async_copy(table_hbm.at[idx_v.at[j]], rows_v.at[j], sem)
            for j in range(n_chunks)
        ]
        for c in copies:
            c.wait()
        for j in range(n_chunks):
            pltpu.sync_copy(rows_v.at[j],
                            out_hbm.at[pl.ds(base + j * _CHUNK, _CHUNK)])

    return k(table, idx_flat)


_ROWS = 256


def _tc_body(g_ref, par_ref, w_ref, tt_ref, tte_ref, pos_ref, gam_ref, bet_ref,
             o_ref):
    g = g_ref[...]
    e = g.shape[1] // 2
    quad = par_ref[0, 0, :]
    h = ((quad >> 1) & 1).astype(jnp.float32)[:, None]
    l = (quad & 1).astype(jnp.float32)[:, None]
    gb = lax.bitcast_convert_type(g, jnp.int32)
    hm = (h * jnp.ones_like(g[:, :e])).astype(jnp.int32)
    sbits = gb[:, :e] + hm * (gb[:, e:] - gb[:, :e])
    hi = lax.bitcast_convert_type(sbits & _HI_MASK, jnp.float32)
    lo = lax.bitcast_convert_type(sbits << 16, jnp.float32)
    rows = hi + l * (lo - hi)
    x = jnp.dot(rows, w_ref[...], preferred_element_type=jnp.float32)
    ttid = tt_ref[0, 0, :].astype(jnp.float32)
    tt = tte_ref[0][None, :] + ttid[:, None] * (tte_ref[1] - tte_ref[0])[None, :]
    x = x + tt + pos_ref[...]
    mean = jnp.mean(x, axis=1, keepdims=True)
    c = x - mean
    var = jnp.mean(c * c, axis=1, keepdims=True)
    o_ref[...] = c * lax.rsqrt(var + LN_EPS) * gam_ref[...] + bet_ref[...]


def _tc_fused(gathered, quad, proj_W, tt_flat, token_type_emb, pos_emb,
              gamma, beta, batch, seq):
    n, e2 = gathered.shape
    h = proj_W.shape[1]
    nb = n // _ROWS
    s_blocks = seq // _ROWS
    tt3 = tt_flat.reshape(nb, 1, _ROWS)
    par3 = quad.reshape(nb, 1, _ROWS)
    out = pl.pallas_call(
        _tc_body,
        grid=(nb,),
        in_specs=[
            pl.BlockSpec((_ROWS, e2), lambda i: (i, 0)),
            pl.BlockSpec((1, 1, _ROWS), lambda i: (i, 0, 0)),
            pl.BlockSpec((e2 // 2, h), lambda i: (0, 0)),
            pl.BlockSpec((1, 1, _ROWS), lambda i: (i, 0, 0)),
            pl.BlockSpec((2, h), lambda i: (0, 0)),
            pl.BlockSpec((_ROWS, h), lambda i: (i % s_blocks, 0)),
            pl.BlockSpec((1, h), lambda i: (0, 0)),
            pl.BlockSpec((1, h), lambda i: (0, 0)),
        ],
        out_specs=pl.BlockSpec((_ROWS, h), lambda i: (i, 0)),
        out_shape=jax.ShapeDtypeStruct((n, h), jnp.float32),
    )(gathered, par3, proj_W, tt3, token_type_emb, pos_emb[:seq],
      gamma.reshape(1, h), beta.reshape(1, h))
    return out.reshape(batch, seq, h)


def kernel(input_ids, token_type_ids, word_emb, proj_W, token_type_emb,
           pos_emb, gamma, beta):
    b, s = input_ids.shape
    idx_flat = input_ids.reshape(-1).astype(jnp.int32)
    tt_flat = token_type_ids.reshape(-1).astype(jnp.int32)
    table2 = _tc_relayout(jnp.swapaxes(word_emb, 0, 1))
    rb = _RBLK.bit_length() - 1
    qb = _QBLK.bit_length() - 1
    quad_row = ((idx_flat >> rb) << qb) | (idx_flat & (_QBLK - 1))
    quad = (idx_flat >> qb) & 3
    gathered = _sc_gather(table2, quad_row)
    return _tc_fused(gathered, quad, proj_W, tt_flat, token_type_emb,
                     pos_emb, gamma, beta, b, s)


# speedup vs baseline: 1.1355x; 1.1355x over previous
import functools

import jax
import jax.numpy as jnp
from jax import lax
from jax.experimental import pallas as pl
from jax.experimental.pallas import tpu as pltpu
from jax.experimental.pallas import tpu_sc as plsc

LN_EPS = 1e-12


_RBLK = 16384


_QBLK = _RBLK // 4
_HI_MASK = -65536


def _round_bf16_bits(t):
    bits = lax.bitcast_convert_type(t, jnp.int32)
    rnd = bits + 0x7FFF + ((bits >> 16) & 1)
    return rnd & _HI_MASK


def _relayout_body(x_ref, o_ref):
    e = x_ref.shape[0]
    for k in range(_QBLK // 128):
        sl = pl.ds(k * 128, 128)
        quads = [
            _round_bf16_bits(jnp.transpose(
                x_ref[:, q * _QBLK + k * 128:q * _QBLK + (k + 1) * 128]))
            for q in range(4)
        ]
        p01 = quads[0] | ((quads[1] >> 16) & 0xFFFF)
        p23 = quads[2] | ((quads[3] >> 16) & 0xFFFF)
        o_ref[sl, :e] = lax.bitcast_convert_type(p01, jnp.float32)
        o_ref[sl, e:] = lax.bitcast_convert_type(p23, jnp.float32)


def _tc_relayout(tableT):
    e, v = tableT.shape
    grid = (v + _RBLK - 1) // _RBLK
    return pl.pallas_call(
        _relayout_body,
        grid=(grid,),
        in_specs=[pl.BlockSpec((e, _RBLK), lambda j: (0, j))],
        out_specs=pl.BlockSpec((_QBLK, 2 * e), lambda j: (j, 0)),
        out_shape=jax.ShapeDtypeStruct((grid * _QBLK, 2 * e), jnp.float32),
        compiler_params=pltpu.CompilerParams(
            dimension_semantics=("parallel",)),
    )(tableT)


_CHUNK = 128


def _sc_gather(table, idx_flat):
    info = plsc.get_sparse_core_info()
    _NC, _NS = info.num_cores, info.num_subcores
    _NW = _NC * _NS
    n, e = idx_flat.shape[0], table.shape[1]
    per_w = n // _NW
    n_chunks = per_w // _CHUNK
    mesh = plsc.VectorSubcoreMesh(core_axis_name="c", subcore_axis_name="s")

    @functools.partial(
        pl.kernel,
        mesh=mesh,
        compiler_params=pltpu.CompilerParams(use_tc_tiling_on_sc=True),
        out_type=jax.ShapeDtypeStruct((n, e), jnp.float32),
        scratch_types=[
            pltpu.VMEM((n_chunks, _CHUNK), jnp.int32),
            pltpu.VMEM((n_chunks, _CHUNK, e), jnp.float32),
            pltpu.SemaphoreType.DMA,
        ],
    )
    def k(table_hbm, idx_hbm, out_hbm, idx_v, rows_v, sem):
        wid = lax.axis_index("s") * _NC + lax.axis_index("c")
        base = wid * per_w
        for j in range(n_chunks):
            pltpu.sync_copy(idx_hbm.at[pl.ds(base + j * _CHUNK, _CHUNK)],
                            idx_v.at[j])
        copies = [
            pltpu.async_copy(table_hbm.at[idx_v.at[j]], rows_v.at[j], sem)
            for j in range(n_chunks)
        ]
        for c in copies:
            c.wait()
        for j in range(n_chunks):
            pltpu.sync_copy(rows_v.at[j],
                            out_hbm.at[pl.ds(base + j * _CHUNK, _CHUNK)])

    return k(table, idx_flat)


_ROWS = 256


def _tc_body(g_ref, par_ref, w_ref, tt_ref, tte_ref, pos_ref, gam_ref, bet_ref,
             o_ref):
    g = g_ref[...]
    e = g.shape[1] // 2
    quad = par_ref[0, 0, :]
    h = ((quad >> 1) & 1).astype(jnp.float32)[:, None]
    l = (quad & 1).astype(jnp.float32)[:, None]
    gb = lax.bitcast_convert_type(g, jnp.int32)
    hm = (h * jnp.ones_like(g[:, :e])).astype(jnp.int32)
    sbits = gb[:, :e] + hm * (gb[:, e:] - gb[:, :e])
    hi = lax.bitcast_convert_type(sbits & _HI_MASK, jnp.float32)
    lo = lax.bitcast_convert_type(sbits << 16, jnp.float32)
    rows = hi + l * (lo - hi)
    x = jnp.dot(rows, w_ref[...], preferred_element_type=jnp.float32)
    ttid = tt_ref[0, 0, :].astype(jnp.float32)
    tt = tte_ref[0][None, :] + ttid[:, None] * (tte_ref[1] - tte_ref[0])[None, :]
    x = x + tt + pos_ref[...]
    mean = jnp.mean(x, axis=1, keepdims=True)
    c = x - mean
    var = jnp.mean(c * c, axis=1, keepdims=True)
    o_ref[...] = c * lax.rsqrt(var + LN_EPS) * gam_ref[...] + bet_ref[...]


def _tc_fused(gathered, quad, proj_W, tt_flat, token_type_emb, pos_emb,
              gamma, beta, batch, seq):
    n, e2 = gathered.shape
    h = proj_W.shape[1]
    nb = n // _ROWS
    s_blocks = seq // _ROWS
    tt3 = tt_flat.reshape(nb, 1, _ROWS)
    par3 = quad.reshape(nb, 1, _ROWS)
    out = pl.pallas_call(
        _tc_body,
        grid=(nb,),
        in_specs=[
            pl.BlockSpec((_ROWS, e2), lambda i: (i, 0)),
            pl.BlockSpec((1, 1, _ROWS), lambda i: (i, 0, 0)),
            pl.BlockSpec((e2 // 2, h), lambda i: (0, 0)),
            pl.BlockSpec((1, 1, _ROWS), lambda i: (i, 0, 0)),
            pl.BlockSpec((2, h), lambda i: (0, 0)),
            pl.BlockSpec((_ROWS, h), lambda i: (i % s_blocks, 0)),
            pl.BlockSpec((1, h), lambda i: (0, 0)),
            pl.BlockSpec((1, h), lambda i: (0, 0)),
        ],
        out_specs=pl.BlockSpec((_ROWS, h), lambda i: (i, 0)),
        out_shape=jax.ShapeDtypeStruct((n, h), jnp.float32),
    )(gathered, par3, proj_W, tt3, token_type_emb, pos_emb[:seq],
      gamma.reshape(1, h), beta.reshape(1, h))
    return out.reshape(batch, seq, h)


def kernel(input_ids, token_type_ids, word_emb, proj_W, token_type_emb,
           pos_emb, gamma, beta):
    b, s = input_ids.shape
    idx_flat = input_ids.reshape(-1).astype(jnp.int32)
    tt_flat = token_type_ids.reshape(-1).astype(jnp.int32)
    table2 = _tc_relayout(jnp.swapaxes(word_emb, 0, 1))
    rb = _RBLK.bit_length() - 1
    qb = _QBLK.bit_length() - 1
    quad_row = ((idx_flat >> rb) << qb) | (idx_flat & (_QBLK - 1))
    quad = (idx_flat >> qb) & 3
    gathered = _sc_gather(table2, quad_row)
    return _tc_fused(gathered, quad, proj_W, tt_flat, token_type_emb,
                     pos_emb, gamma, beta, b, s)


# speedup vs baseline: 1.1870x; 1.0454x over previous
import functools

import jax
import jax.numpy as jnp
from jax import lax
from jax.experimental import pallas as pl
from jax.experimental.pallas import tpu as pltpu
from jax.experimental.pallas import tpu_sc as plsc

LN_EPS = 1e-12


_RBLK = 32768


_QBLK = _RBLK // 4
_HI_MASK = -65536


def _round_bf16_bits(t):
    bits = lax.bitcast_convert_type(t, jnp.int32)
    rnd = bits + 0x7FFF + ((bits >> 16) & 1)
    return rnd & _HI_MASK


def _relayout_body(x_ref, o_ref):
    e = x_ref.shape[0]
    for k in range(_QBLK // 128):
        sl = pl.ds(k * 128, 128)
        quads = [
            _round_bf16_bits(jnp.transpose(
                x_ref[:, q * _QBLK + k * 128:q * _QBLK + (k + 1) * 128]))
            for q in range(4)
        ]
        p01 = quads[0] | ((quads[1] >> 16) & 0xFFFF)
        p23 = quads[2] | ((quads[3] >> 16) & 0xFFFF)
        o_ref[sl, :e] = lax.bitcast_convert_type(p01, jnp.float32)
        o_ref[sl, e:] = lax.bitcast_convert_type(p23, jnp.float32)


def _tc_relayout(tableT):
    e, v = tableT.shape
    grid = (v + _RBLK - 1) // _RBLK
    return pl.pallas_call(
        _relayout_body,
        grid=(grid,),
        in_specs=[pl.BlockSpec((e, _RBLK), lambda j: (0, j))],
        out_specs=pl.BlockSpec((_QBLK, 2 * e), lambda j: (j, 0)),
        out_shape=jax.ShapeDtypeStruct((grid * _QBLK, 2 * e), jnp.float32),
        compiler_params=pltpu.CompilerParams(
            dimension_semantics=("parallel",)),
    )(tableT)


_CHUNK = 128


def _sc_gather(table, idx_flat):
    info = plsc.get_sparse_core_info()
    _NC, _NS = info.num_cores, info.num_subcores
    _NW = _NC * _NS
    n, e = idx_flat.shape[0], table.shape[1]
    per_w = n // _NW
    n_chunks = per_w // _CHUNK
    mesh = plsc.VectorSubcoreMesh(core_axis_name="c", subcore_axis_name="s")

    @functools.partial(
        pl.kernel,
        mesh=mesh,
        compiler_params=pltpu.CompilerParams(use_tc_tiling_on_sc=True),
        out_type=jax.ShapeDtypeStruct((n, e), jnp.float32),
        scratch_types=[
            pltpu.VMEM((n_chunks, _CHUNK), jnp.int32),
            pltpu.VMEM((n_chunks, _CHUNK, e), jnp.float32),
            pltpu.SemaphoreType.DMA,
        ],
    )
    def k(table_hbm, idx_hbm, out_hbm, idx_v, rows_v, sem):
        wid = lax.axis_index("s") * _NC + lax.axis_index("c")
        base = wid * per_w
        for j in range(n_chunks):
            pltpu.sync_copy(idx_hbm.at[pl.ds(base + j * _CHUNK, _CHUNK)],
                            idx_v.at[j])
        copies = [
            pltpu.async_copy(table_hbm.at[idx_v.at[j]], rows_v.at[j], sem)
            for j in range(n_chunks)
        ]
        for c in copies:
            c.wait()
        for j in range(n_chunks):
            pltpu.sync_copy(rows_v.at[j],
                            out_hbm.at[pl.ds(base + j * _CHUNK, _CHUNK)])

    return k(table, idx_flat)


_ROWS = 256


def _tc_body(g_ref, par_ref, w_ref, tt_ref, tte_ref, pos_ref, gam_ref, bet_ref,
             o_ref):
    g = g_ref[...]
    e = g.shape[1] // 2
    quad = par_ref[0, 0, :]
    h = ((quad >> 1) & 1).astype(jnp.float32)[:, None]
    l = (quad & 1).astype(jnp.float32)[:, None]
    gb = lax.bitcast_convert_type(g, jnp.int32)
    hm = (h * jnp.ones_like(g[:, :e])).astype(jnp.int32)
    sbits = gb[:, :e] + hm * (gb[:, e:] - gb[:, :e])
    hi = lax.bitcast_convert_type(sbits & _HI_MASK, jnp.float32)
    lo = lax.bitcast_convert_type(sbits << 16, jnp.float32)
    rows = hi + l * (lo - hi)
    x = jnp.dot(rows, w_ref[...], preferred_element_type=jnp.float32)
    ttid = tt_ref[0, 0, :].astype(jnp.float32)
    tt = tte_ref[0][None, :] + ttid[:, None] * (tte_ref[1] - tte_ref[0])[None, :]
    x = x + tt + pos_ref[...]
    mean = jnp.mean(x, axis=1, keepdims=True)
    c = x - mean
    var = jnp.mean(c * c, axis=1, keepdims=True)
    o_ref[...] = c * lax.rsqrt(var + LN_EPS) * gam_ref[...] + bet_ref[...]


def _tc_fused(gathered, quad, proj_W, tt_flat, token_type_emb, pos_emb,
              gamma, beta, batch, seq):
    n, e2 = gathered.shape
    h = proj_W.shape[1]
    nb = n // _ROWS
    s_blocks = seq // _ROWS
    tt3 = tt_flat.reshape(nb, 1, _ROWS)
    par3 = quad.reshape(nb, 1, _ROWS)
    out = pl.pallas_call(
        _tc_body,
        grid=(nb,),
        in_specs=[
            pl.BlockSpec((_ROWS, e2), lambda i: (i, 0)),
            pl.BlockSpec((1, 1, _ROWS), lambda i: (i, 0, 0)),
            pl.BlockSpec((e2 // 2, h), lambda i: (0, 0)),
            pl.BlockSpec((1, 1, _ROWS), lambda i: (i, 0, 0)),
            pl.BlockSpec((2, h), lambda i: (0, 0)),
            pl.BlockSpec((_ROWS, h), lambda i: (i % s_blocks, 0)),
            pl.BlockSpec((1, h), lambda i: (0, 0)),
            pl.BlockSpec((1, h), lambda i: (0, 0)),
        ],
        out_specs=pl.BlockSpec((_ROWS, h), lambda i: (i, 0)),
        out_shape=jax.ShapeDtypeStruct((n, h), jnp.float32),
    )(gathered, par3, proj_W, tt3, token_type_emb, pos_emb[:seq],
      gamma.reshape(1, h), beta.reshape(1, h))
    return out.reshape(batch, seq, h)


def kernel(input_ids, token_type_ids, word_emb, proj_W, token_type_emb,
           pos_emb, gamma, beta):
    b, s = input_ids.shape
    idx_flat = input_ids.reshape(-1).astype(jnp.int32)
    tt_flat = token_type_ids.reshape(-1).astype(jnp.int32)
    table2 = _tc_relayout(jnp.swapaxes(word_emb, 0, 1))
    rb = _RBLK.bit_length() - 1
    qb = _QBLK.bit_length() - 1
    quad_row = ((idx_flat >> rb) << qb) | (idx_flat & (_QBLK - 1))
    quad = (idx_flat >> qb) & 3
    gathered = _sc_gather(table2, quad_row)
    return _tc_fused(gathered, quad, proj_W, tt_flat, token_type_emb,
                     pos_emb, gamma, beta, b, s)
